# baseline (speedup 1.0000x reference)
import jax
import jax.numpy as jnp
from jax.experimental import pallas as pl

N = 10000
B = 64
RATIO = 0.8


def _mlp_body(z_ref, lw1_ref, lb1_ref, lw2_ref, lb2_ref, lw3_ref, lb3_ref,
              lw4_ref, lb4_ref, lw5_ref, lb5_ref, o_ref):
    z = z_ref[...]
    z = jax.nn.relu(z @ lw1_ref[...] + lb1_ref[...])
    z = jax.nn.relu(z @ lw2_ref[...] + lb2_ref[...])
    z = jax.nn.relu(z @ lw3_ref[...] + lb3_ref[...])
    z = jax.nn.relu(z @ lw4_ref[...] + lb4_ref[...])
    o_ref[...] = jax.nn.sigmoid(z @ lw5_ref[...] + lb5_ref[...])


def _counting_topk(score, batch, mask, prev, starts):
    neg = jnp.where(mask, -score, jnp.inf)
    same = batch[None, :] == batch[:, None]
    lt = (neg[None, :] < neg[:, None]) | (
        (neg[None, :] == neg[:, None]) & (prev[None, :] < prev[:, None]))
    rank = jnp.sum(same & lt, axis=1).astype(jnp.int32)
    pos = starts[batch] + rank
    kept = jax.ops.segment_sum(mask.astype(jnp.int32), batch, num_segments=B)
    k = jnp.ceil(RATIO * kept.astype(jnp.float32)).astype(jnp.int32)
    keep = mask & (rank < k[batch])
    return keep, pos


def _gmp_gap(h, b, mask):
    m = mask.astype(h.dtype)
    s = jax.ops.segment_sum(h * m[:, None], b, num_segments=B)
    cnt = jax.ops.segment_sum(m, b, num_segments=B)
    mean = s / jnp.maximum(cnt, 1.0)[:, None]
    mx = jax.ops.segment_max(jnp.where(mask[:, None], h, -jnp.inf), b, num_segments=B)
    mx = jnp.where(cnt[:, None] > 0, mx, 0.0)
    return jnp.concatenate([mx, mean], axis=1)


def kernel(emb, w_l1, b_l1, w_r1, w_l2, b_l2, w_r2, w_l3, b_l3, w_r3,
           p1, p2, p3, lw1, lb1, lw2, lb2, lw3, lb3, lw4, lb4, lw5, lb5,
           x, edge_index, batch):
    ws = {1: (w_l1, b_l1, w_r1, p1), 2: (w_l2, b_l2, w_r2, p2), 3: (w_l3, b_l3, w_r3, p3)}
    src, dst = edge_index[0], edge_index[1]
    n = batch.shape[0]
    h = emb[x[:, 0]]
    counts = jax.ops.segment_sum(jnp.ones((n,), jnp.int32), batch, num_segments=B)
    starts = jnp.concatenate([jnp.zeros((1,), jnp.int32), jnp.cumsum(counts)[:-1]])
    mask = jnp.ones((n,), bool)
    keepf = jnp.ones((n,), jnp.float32)
    prev = jnp.arange(n, dtype=jnp.int32)
    reads = []
    for i in (1, 2, 3):
        w_l, b_l, w_r, p = ws[i]
        ewe = keepf[src] * keepf[dst]
        agg = jax.ops.segment_sum(h[src] * ewe[:, None], dst, num_segments=n)
        deg = jax.ops.segment_sum(ewe, dst, num_segments=n)
        agg = agg / jnp.maximum(deg, 1.0)[:, None]
        h = jax.nn.relu(agg @ w_l + b_l + h @ w_r)
        score = jnp.tanh((h @ p) / jnp.linalg.norm(p))
        keep, pos = _counting_topk(score, batch, mask, prev, starts)
        h = jnp.where(keep[:, None], h * score[:, None], 0.0)
        reads.append(_gmp_gap(h, batch, keep))
        mask, prev = keep, pos
        keepf = keep.astype(jnp.float32)

    z = reads[0] + reads[1] + reads[2]
    out = pl.pallas_call(
        _mlp_body,
        out_shape=jax.ShapeDtypeStruct((B, 1), jnp.float32),
    )(z, lw1, lb1, lw2, lb2, lw3, lb3, lw4, lb4, lw5, lb5)
    return out[:, 0]


# speedup vs baseline: 1.0321x; 1.0321x over previous
import functools
import jax
import jax.numpy as jnp
from jax.experimental import pallas as pl
from jax.experimental.pallas import tpu as pltpu

N = 10000
NP = 10240
BLK = 1024
NB = NP // BLK
B = 64
RATIO = 0.8
NEGINF = -jnp.inf


def _cnt_body(bt_ref, cs_ref, st_ref):
    i = pl.program_id(0)

    @pl.when(i == 0)
    def _():
        cs_ref[...] = jnp.zeros_like(cs_ref)

    bt = bt_ref[pl.ds(i * BLK, BLK), :]
    gi = jax.lax.broadcasted_iota(jnp.int32, (1, 128), 1)
    mb = (bt == gi).astype(jnp.float32)
    cs_ref[...] += jnp.sum(mb, axis=0)[:, None]

    @pl.when(i == NB - 1)
    def _():
        r = jax.lax.broadcasted_iota(jnp.int32, (128, 128), 0)
        c = jax.lax.broadcasted_iota(jnp.int32, (128, 128), 1)
        ltri = (c < r).astype(jnp.float32)
        st_ref[...] = jnp.dot(ltri, cs_ref[...],
                              preferred_element_type=jnp.float32)


def _k_cnt(batchi):
    return pl.pallas_call(
        _cnt_body,
        grid=(NB,),
        in_specs=[pl.BlockSpec((NP, 1), lambda i: (0, 0))],
        out_specs=[pl.BlockSpec((128, 1), lambda i: (0, 0)),
                   pl.BlockSpec((128, 1), lambda i: (0, 0))],
        out_shape=[jax.ShapeDtypeStruct((128, 1), jnp.float32),
                   jax.ShapeDtypeStruct((128, 1), jnp.float32)],
    )(batchi)


def _dense_body(h_ref, agg_ref, deg_ref, wl_ref, bl_ref, wr_ref, p_ref,
                h1_ref, sc_ref):
    a = agg_ref[...] / jnp.maximum(deg_ref[...], 1.0)
    h1 = jnp.dot(a, wl_ref[...], preferred_element_type=jnp.float32)
    h1 += jnp.dot(h_ref[...], wr_ref[...], preferred_element_type=jnp.float32)
    h1 = jnp.maximum(h1 + bl_ref[...], 0.0)
    h1_ref[...] = h1
    pv = p_ref[...]
    nrm = jnp.sqrt(jnp.sum(pv * pv))
    sc_ref[...] = jnp.tanh(
        jnp.dot(h1, pv, preferred_element_type=jnp.float32) / nrm)


def _k_dense(h, agg, deg, w_l, b_l, w_r, p):
    din = h.shape[1]
    return pl.pallas_call(
        _dense_body,
        grid=(NB,),
        in_specs=[
            pl.BlockSpec((BLK, din), lambda i: (i, 0)),
            pl.BlockSpec((BLK, din), lambda i: (i, 0)),
            pl.BlockSpec((BLK, 1), lambda i: (i, 0)),
            pl.BlockSpec((din, 512), lambda i: (0, 0)),
            pl.BlockSpec((1, 512), lambda i: (0, 0)),
            pl.BlockSpec((din, 512), lambda i: (0, 0)),
            pl.BlockSpec((512, 1), lambda i: (0, 0)),
        ],
        out_specs=[pl.BlockSpec((BLK, 512), lambda i: (i, 0)),
                   pl.BlockSpec((BLK, 1), lambda i: (i, 0))],
        out_shape=[jax.ShapeDtypeStruct((NP, 512), jnp.float32),
                   jax.ShapeDtypeStruct((NP, 1), jnp.float32)],
    )(h, agg, deg, w_l, b_l[None, :], w_r, p[:, None])


def _topk_body(sc_col_ref, bt_col_ref, mk_col_ref, pv_col_ref,
               sc_row_ref, bt_row_ref, mk_row_ref, pv_row_ref,
               cs_ref, st_ref,
               keep_ref, pos_ref, cnt_ref,
               kept_s, k_s):
    pid = pl.program_id(0)
    gi64 = jax.lax.broadcasted_iota(jnp.int32, (1, B), 1)

    @pl.when(pid == 0)
    def _():
        kept_s[...] = jnp.zeros_like(kept_s)

    @pl.when(pid < NB)
    def _():
        bt = bt_col_ref[pl.ds(pid * BLK, BLK), :]
        m = mk_col_ref[pl.ds(pid * BLK, BLK), :]
        mb = (bt == gi64).astype(jnp.float32)
        kept_s[...] += jnp.sum(mb * m, axis=0)[:, None]

    @pl.when(pid == NB - 1)
    def _():
        k_s[...] = jnp.ceil(jnp.float32(RATIO) * kept_s[...])

    @pl.when(pid == NB)
    def _():
        cnt_ref[...] = jnp.zeros_like(cnt_ref)

    @pl.when(pid >= NB)
    def _():
        i2 = pid - NB
        r0 = i2 * BLK
        bt_r = bt_col_ref[pl.ds(r0, BLK), :]
        mk_r = mk_col_ref[pl.ds(r0, BLK), :]
        sc_r = sc_col_ref[pl.ds(r0, BLK), :]
        pv_r = pv_col_ref[pl.ds(r0, BLK), :]
        neg_r = jnp.where(mk_r > 0, -sc_r, jnp.inf)

        def col_chunk(c, rank):
            bt_c = bt_row_ref[:, pl.ds(c * BLK, BLK)]
            mk_c = mk_row_ref[:, pl.ds(c * BLK, BLK)]
            sc_c = sc_row_ref[:, pl.ds(c * BLK, BLK)]
            pv_c = pv_row_ref[:, pl.ds(c * BLK, BLK)]
            neg_c = jnp.where(mk_c > 0, -sc_c, jnp.inf)
            same = bt_c == bt_r
            lt = (neg_c < neg_r) | ((neg_c == neg_r) & (pv_c < pv_r))
            return rank + jnp.sum((same & lt).astype(jnp.float32),
                                  axis=1, keepdims=True)

        g_lo = bt_col_ref[r0, 0]
        g_hi = bt_col_ref[r0 + BLK - 1, 0]
        c_lo = st_ref[g_lo, 0].astype(jnp.int32) // BLK
        e_col = (st_ref[g_hi, 0] + cs_ref[g_hi, 0]).astype(jnp.int32)
        c_hi = (e_col + BLK - 1) // BLK
        rank = jax.lax.fori_loop(c_lo, c_hi,
                                 col_chunk, jnp.zeros((BLK, 1), jnp.float32))

        mb_r = (bt_r == gi64).astype(jnp.float32)
        k_r = jnp.dot(mb_r, k_s[...], preferred_element_type=jnp.float32)
        keep_r = mk_r * (rank < k_r).astype(jnp.float32)
        st_r = jnp.dot(mb_r, st_ref[0:B, :],
                       preferred_element_type=jnp.float32)
        keep_ref[pl.ds(r0, BLK), :] = keep_r
        pos_ref[pl.ds(r0, BLK), :] = st_r + rank
        cnt_ref[...] += jnp.sum(mb_r * keep_r, axis=0)[:, None]


def _k_topk(sc_col, bt_col, mk_col, pv_col, cs, st):
    full = lambda shape: pl.BlockSpec(shape, lambda i: (0, 0))
    return pl.pallas_call(
        _topk_body,
        grid=(2 * NB,),
        in_specs=[full((NP, 1)), full((NP, 1)), full((NP, 1)), full((NP, 1)),
                  full((1, NP)), full((1, NP)), full((1, NP)), full((1, NP)),
                  full((128, 1)), full((128, 1))],
        out_specs=[full((NP, 1)), full((NP, 1)), full((B, 1))],
        out_shape=[jax.ShapeDtypeStruct((NP, 1), jnp.float32),
                   jax.ShapeDtypeStruct((NP, 1), jnp.float32),
                   jax.ShapeDtypeStruct((B, 1), jnp.float32)],
        scratch_shapes=[pltpu.VMEM((B, 1), jnp.float32),
                        pltpu.VMEM((B, 1), jnp.float32)],
    )(sc_col, bt_col, mk_col, pv_col,
      sc_col.reshape(1, NP), bt_col.reshape(1, NP),
      mk_col.reshape(1, NP), pv_col.reshape(1, NP), cs, st)


def _pool_body(h1_ref, sc_ref, keep_ref, bt_ref, cnt_ref,
               h2_ref, read_ref, sum_s, max_s):
    i = pl.program_id(0)

    @pl.when(i == 0)
    def _():
        sum_s[...] = jnp.zeros_like(sum_s)
        max_s[...] = jnp.full_like(max_s, NEGINF)

    keep = keep_ref[...]
    h2 = h1_ref[...] * (sc_ref[...] * keep)
    h2_ref[...] = h2
    bt = bt_ref[...]
    gi64 = jax.lax.broadcasted_iota(jnp.int32, (1, B), 1)
    mk = (bt == gi64).astype(jnp.float32) * keep
    sum_s[...] += jax.lax.dot_general(
        mk, h2, (((0,), (0,)), ((), ())),
        preferred_element_type=jnp.float32)

    g_lo = bt[0, 0]
    g_hi = jnp.minimum(bt[BLK - 1, 0], B - 1)

    def upd(g, _):
        rows = (bt == g) & (keep > 0)
        cand = jnp.max(jnp.where(rows, h2, NEGINF), axis=0, keepdims=True)
        cur = max_s[pl.ds(g, 1), :]
        max_s[pl.ds(g, 1), :] = jnp.maximum(cur, cand)
        return 0

    jax.lax.fori_loop(g_lo, g_hi + 1, upd, 0)

    @pl.when(i == NB - 1)
    def _():
        cnt = cnt_ref[...]
        read_ref[:, 0:512] = jnp.where(cnt > 0, max_s[...], 0.0)
        read_ref[:, 512:1024] = sum_s[...] / jnp.maximum(cnt, 1.0)


def _k_pool(h1, sc, keep, bt, cnt):
    full = lambda shape: pl.BlockSpec(shape, lambda i: (0, 0))
    return pl.pallas_call(
        _pool_body,
        grid=(NB,),
        in_specs=[pl.BlockSpec((BLK, 512), lambda i: (i, 0)),
                  pl.BlockSpec((BLK, 1), lambda i: (i, 0)),
                  pl.BlockSpec((BLK, 1), lambda i: (i, 0)),
                  pl.BlockSpec((BLK, 1), lambda i: (i, 0)),
                  full((B, 1))],
        out_specs=[pl.BlockSpec((BLK, 512), lambda i: (i, 0)),
                   full((B, 1024))],
        out_shape=[jax.ShapeDtypeStruct((NP, 512), jnp.float32),
                   jax.ShapeDtypeStruct((B, 1024), jnp.float32)],
        scratch_shapes=[pltpu.VMEM((B, 512), jnp.float32),
                        pltpu.VMEM((B, 512), jnp.float32)],
    )(h1, sc, keep, bt, cnt)


def _mlp_body(r1_ref, r2_ref, r3_ref, w1, b1, w2, b2, w3, b3, w4, b4, w5, b5,
              o_ref):
    z = r1_ref[...] + r2_ref[...] + r3_ref[...]
    z = jnp.maximum(jnp.dot(z, w1[...], preferred_element_type=jnp.float32)
                    + b1[...], 0.0)
    z = jnp.maximum(jnp.dot(z, w2[...], preferred_element_type=jnp.float32)
                    + b2[...], 0.0)
    z = jnp.maximum(jnp.dot(z, w3[...], preferred_element_type=jnp.float32)
                    + b3[...], 0.0)
    z = jnp.maximum(jnp.dot(z, w4[...], preferred_element_type=jnp.float32)
                    + b4[...], 0.0)
    z = jnp.dot(z, w5[...], preferred_element_type=jnp.float32) + b5[...]
    o_ref[...] = 1.0 / (1.0 + jnp.exp(-z))


def _k_mlp(r1, r2, r3, lw1, lb1, lw2, lb2, lw3, lb3, lw4, lb4, lw5, lb5):
    return pl.pallas_call(
        _mlp_body,
        out_shape=jax.ShapeDtypeStruct((B, 1), jnp.float32),
    )(r1, r2, r3, lw1, lb1[None, :], lw2, lb2[None, :], lw3, lb3[None, :],
      lw4, lb4[None, :], lw5, lb5[None, :])


def kernel(emb, w_l1, b_l1, w_r1, w_l2, b_l2, w_r2, w_l3, b_l3, w_r3,
           p1, p2, p3, lw1, lb1, lw2, lb2, lw3, lb3, lw4, lb4, lw5, lb5,
           x, edge_index, batch):
    ws = {1: (w_l1, b_l1, w_r1, p1), 2: (w_l2, b_l2, w_r2, p2),
          3: (w_l3, b_l3, w_r3, p3)}
    src, dst = edge_index[0], edge_index[1]

    h = emb[x[:, 0]]
    h = jnp.pad(h, ((0, NP - N), (0, 0)))
    batchi = jnp.pad(batch.astype(jnp.int32), (0, NP - N),
                     constant_values=B)[:, None]
    cs, st = _k_cnt(batchi)

    maskf = jnp.pad(jnp.ones((N,), jnp.float32), (0, NP - N))[:, None]
    prevf = jnp.arange(NP, dtype=jnp.float32)[:, None]
    keep_flat = jnp.ones((N,), jnp.float32)

    reads = []
    for i in (1, 2, 3):
        w_l, b_l, w_r, p = ws[i]
        ewe = keep_flat[src] * keep_flat[dst]
        hn = h[:N]
        aggsum = jax.ops.segment_sum(hn[src] * ewe[:, None], dst,
                                     num_segments=N)
        deg = jax.ops.segment_sum(ewe, dst, num_segments=N)
        agg = jnp.pad(aggsum, ((0, NP - N), (0, 0)))
        degp = jnp.pad(deg, (0, NP - N))[:, None]

        h1, sc = _k_dense(h, agg, degp, w_l, b_l, w_r, p)
        keepf, posf, cnt = _k_topk(sc, batchi, maskf, prevf, cs, st)
        h, read = _k_pool(h1, sc, keepf, batchi, cnt)
        reads.append(read)
        maskf, prevf = keepf, posf
        keep_flat = keepf[:N, 0]

    out = _k_mlp(reads[0], reads[1], reads[2],
                 lw1, lb1, lw2, lb2, lw3, lb3, lw4, lb4, lw5, lb5)
    return out[:, 0]


# speedup vs baseline: 4.5922x; 4.4493x over previous
import functools
import jax
import jax.numpy as jnp
from jax import lax
from jax.experimental import pallas as pl
from jax.experimental.pallas import tpu as pltpu
from jax.experimental.pallas import tpu_sc as plsc

N = 10000
NP = 10240
BLK = 1024
NB = NP // BLK
B = 64
RATIO = 0.8
NEGINF = -jnp.inf

NC, NS, L = 2, 16, 16
E = 320000
EPT = E // NS
ECH = 2000
NCHUNK = EPT // ECH
GCAP = EPT + 160


def _sc_mesh():
    return plsc.VectorSubcoreMesh(core_axis_name="c", subcore_axis_name="s",
                                  num_cores=NC, num_subcores=NS)


def _emb_body(emb_hbm, xi_hbm, out_hbm, idx_v, rows_v, sem):
    c = lax.axis_index("c")
    s = lax.axis_index("s")
    wid = s * NC + c
    bpw = NP // (NC * NS)
    base = wid * bpw
    pltpu.sync_copy(xi_hbm.at[pl.ds(base, bpw)], idx_v)
    for k in range(bpw // 64):
        pltpu.async_copy(emb_hbm.at[idx_v.at[pl.ds(k * 64, 64)]],
                         rows_v.at[pl.ds(k * 64, 64)], sem).wait()
    pltpu.sync_copy(rows_v, out_hbm.at[pl.ds(base, bpw)])


def _k_emb(emb, xi):
    bpw = NP // (NC * NS)
    return pl.kernel(
        _emb_body,
        out_type=jax.ShapeDtypeStruct((NP, 128), jnp.float32),
        mesh=_sc_mesh(),
        scratch_types=[pltpu.VMEM((bpw,), jnp.int32),
                       pltpu.VMEM((bpw, 128), jnp.float32),
                       pltpu.SemaphoreType.DMA],
    )(emb, xi)


def _make_agg_body(din, daug, slice_rows, npass):
    def body(h_hbm, src_hbm, dst_hbm, keep_hbm, eo_hbm, agg_hbm,
             keep_v, eo_v, src_v, dst_v, gidx, sidx, rows_v, acc_v, sem):
        c = lax.axis_index("c")
        s = lax.axis_index("s")
        wid = s * NC + c
        pltpu.sync_copy(keep_hbm, keep_v)
        pltpu.sync_copy(eo_hbm, eo_v)

        for p in range(npass):
            k = p * (NC * NS) + wid
            base = k * slice_rows

            def zr(r, _):
                for cc in range(daug // L):
                    acc_v[r, pl.ds(cc * L, L)] = jnp.zeros((L,), jnp.float32)
                return 0
            lax.fori_loop(0, slice_rows, zr, 0)

            eo16 = eo_v[pl.ds(k, L)]
            e0 = eo16[0]
            e1 = eo16[1]
            a0 = (e0 // 8) * 8
            nch = (e1 - a0 + ECH - 1) // ECH

            def chunk(ch, _):
                cb = a0 + ch * ECH
                pltpu.sync_copy(src_hbm.at[pl.ds(cb, ECH)], src_v)
                pltpu.sync_copy(dst_hbm.at[pl.ds(cb, ECH)], dst_v)

                def scan16(j, cnt):
                    s16 = src_v[pl.ds(j * L, L)]
                    d16 = dst_v[pl.ds(j * L, L)]
                    ks = plsc.load_gather(keep_v, [s16])
                    kd = plsc.load_gather(keep_v, [d16])
                    m = ((d16 >= base) & (d16 < base + slice_rows)
                         & (ks > 0.0) & (kd > 0.0))
                    plsc.store_compressed(gidx.at[pl.ds(cnt, L)], s16, mask=m)
                    plsc.store_compressed(sidx.at[pl.ds(cnt, L)], d16 - base,
                                          mask=m)
                    return cnt + jnp.max(
                        plsc.all_reduce_population_count(m))
                cnt = lax.fori_loop(0, ECH // L, scan16, jnp.int32(0))

                for t in range(2):
                    gidx[pl.ds(cnt + t * L, L)] = jnp.zeros((L,), jnp.int32)
                nsg = (cnt + 31) // 32

                def fl(sg, _):
                    cps = []
                    for j in range(2):
                        i16 = gidx[pl.ds((sg * 2 + j) * L, L)]
                        cps.append(pltpu.async_copy(
                            h_hbm.at[i16], rows_v.at[pl.ds(j * L, L)], sem))
                    for cp in cps:
                        cp.wait()

                    def acc_row(r, _):
                        slot = r - sg * 32
                        dl = sidx[pl.ds(r, L)][0]
                        for cc in range(daug // L):
                            plsc.addupdate(
                                acc_v.at[dl, pl.ds(cc * L, L)],
                                rows_v[slot, pl.ds(cc * L, L)])
                        return 0
                    lax.fori_loop(sg * 32,
                                  jnp.minimum((sg + 1) * 32, cnt),
                                  acc_row, 0)
                    return 0
                lax.fori_loop(0, nsg, fl, 0)
                return 0
            lax.fori_loop(0, nch, chunk, 0)

            @pl.when(base < NP)
            def _():
                pltpu.sync_copy(acc_v,
                                agg_hbm.at[pl.ds(base, slice_rows)])
    return body


def _k_agg(haug, src_s, dst_s, keepf, eo, din, daug, slice_rows, npass):
    body = _make_agg_body(din, daug, slice_rows, npass)
    return pl.kernel(
        body,
        out_type=jax.ShapeDtypeStruct((NP, daug), jnp.float32),
        mesh=_sc_mesh(),
        compiler_params=pltpu.CompilerParams(needs_layout_passes=False),
        scratch_types=[pltpu.VMEM((NP,), jnp.float32),
                       pltpu.VMEM((128,), jnp.int32),
                       pltpu.VMEM((ECH,), jnp.int32),
                       pltpu.VMEM((ECH,), jnp.int32),
                       pltpu.VMEM((ECH + 64,), jnp.int32),
                       pltpu.VMEM((ECH + 64,), jnp.int32),
                       pltpu.VMEM((32, daug), jnp.float32),
                       pltpu.VMEM((slice_rows, daug), jnp.float32),
                       pltpu.SemaphoreType.DMA],
    )(haug, src_s, dst_s, keepf, eo)


def _cnt_body(bt_ref, cs_ref, st_ref):
    i = pl.program_id(0)

    @pl.when(i == 0)
    def _():
        cs_ref[...] = jnp.zeros_like(cs_ref)

    bt = bt_ref[pl.ds(i * BLK, BLK), :]
    gi = jax.lax.broadcasted_iota(jnp.int32, (1, 128), 1)
    mb = (bt == gi).astype(jnp.float32)
    cs_ref[...] += jnp.sum(mb, axis=0)[:, None]

    @pl.when(i == NB - 1)
    def _():
        r = jax.lax.broadcasted_iota(jnp.int32, (128, 128), 0)
        c = jax.lax.broadcasted_iota(jnp.int32, (128, 128), 1)
        ltri = (c < r).astype(jnp.float32)
        st_ref[...] = jnp.dot(ltri, cs_ref[...],
                              preferred_element_type=jnp.float32)


def _k_cnt(batchi):
    return pl.pallas_call(
        _cnt_body,
        grid=(NB,),
        in_specs=[pl.BlockSpec((NP, 1), lambda i: (0, 0))],
        out_specs=[pl.BlockSpec((128, 1), lambda i: (0, 0)),
                   pl.BlockSpec((128, 1), lambda i: (0, 0))],
        out_shape=[jax.ShapeDtypeStruct((128, 1), jnp.float32),
                   jax.ShapeDtypeStruct((128, 1), jnp.float32)],
    )(batchi)


def _dense_body(din, h_ref, agg_ref, wl_ref, bl_ref, wr_ref, p_ref,
                h1_ref, sc_ref):
    agg = agg_ref[:, 0:din]
    deg = agg_ref[:, din:din + 1]
    a = agg / jnp.maximum(deg, 1.0)
    h1 = jnp.dot(a, wl_ref[...], preferred_element_type=jnp.float32)
    h1 += jnp.dot(h_ref[:, 0:din], wr_ref[...],
                  preferred_element_type=jnp.float32)
    h1 = jnp.maximum(h1 + bl_ref[...], 0.0)
    h1_ref[...] = h1
    pv = p_ref[...]
    nrm = jnp.sqrt(jnp.sum(pv * pv))
    sc_ref[...] = jnp.tanh(
        jnp.dot(h1, pv, preferred_element_type=jnp.float32) / nrm)


def _k_dense(haug, aggaug, w_l, b_l, w_r, p, din):
    daug = din + 128
    return pl.pallas_call(
        functools.partial(_dense_body, din),
        grid=(NB,),
        in_specs=[
            pl.BlockSpec((BLK, daug), lambda i: (i, 0)),
            pl.BlockSpec((BLK, daug), lambda i: (i, 0)),
            pl.BlockSpec((din, 512), lambda i: (0, 0)),
            pl.BlockSpec((1, 512), lambda i: (0, 0)),
            pl.BlockSpec((din, 512), lambda i: (0, 0)),
            pl.BlockSpec((512, 1), lambda i: (0, 0)),
        ],
        out_specs=[pl.BlockSpec((BLK, 512), lambda i: (i, 0)),
                   pl.BlockSpec((BLK, 1), lambda i: (i, 0))],
        out_shape=[jax.ShapeDtypeStruct((NP, 512), jnp.float32),
                   jax.ShapeDtypeStruct((NP, 1), jnp.float32)],
    )(haug, aggaug, w_l, b_l[None, :], w_r, p[:, None])


def _topk_body(sc_col_ref, bt_col_ref, mk_col_ref, pv_col_ref,
               sc_row_ref, bt_row_ref, mk_row_ref, pv_row_ref,
               cs_ref, st_ref,
               keep_ref, pos_ref, cnt_ref,
               kept_s, k_s):
    pid = pl.program_id(0)
    gi64 = jax.lax.broadcasted_iota(jnp.int32, (1, B), 1)

    @pl.when(pid == 0)
    def _():
        kept_s[...] = jnp.zeros_like(kept_s)

    @pl.when(pid < NB)
    def _():
        bt = bt_col_ref[pl.ds(pid * BLK, BLK), :]
        m = mk_col_ref[pl.ds(pid * BLK, BLK), :]
        mb = (bt == gi64).astype(jnp.float32)
        kept_s[...] += jnp.sum(mb * m, axis=0)[:, None]

    @pl.when(pid == NB - 1)
    def _():
        k_s[...] = jnp.ceil(jnp.float32(RATIO) * kept_s[...])

    @pl.when(pid == NB)
    def _():
        cnt_ref[...] = jnp.zeros_like(cnt_ref)

    @pl.when(pid >= NB)
    def _():
        i2 = pid - NB
        r0 = i2 * BLK
        bt_r = bt_col_ref[pl.ds(r0, BLK), :]
        mk_r = mk_col_ref[pl.ds(r0, BLK), :]
        sc_r = sc_col_ref[pl.ds(r0, BLK), :]
        pv_r = pv_col_ref[pl.ds(r0, BLK), :]
        neg_r = jnp.where(mk_r > 0, -sc_r, jnp.inf)

        def col_chunk(c, rank):
            bt_c = bt_row_ref[:, pl.ds(c * BLK, BLK)]
            mk_c = mk_row_ref[:, pl.ds(c * BLK, BLK)]
            sc_c = sc_row_ref[:, pl.ds(c * BLK, BLK)]
            pv_c = pv_row_ref[:, pl.ds(c * BLK, BLK)]
            neg_c = jnp.where(mk_c > 0, -sc_c, jnp.inf)
            same = bt_c == bt_r
            lt = (neg_c < neg_r) | ((neg_c == neg_r) & (pv_c < pv_r))
            return rank + jnp.sum((same & lt).astype(jnp.float32),
                                  axis=1, keepdims=True)

        g_lo = bt_col_ref[r0, 0]
        g_hi = bt_col_ref[r0 + BLK - 1, 0]
        c_lo = st_ref[g_lo, 0].astype(jnp.int32) // BLK
        e_col = (st_ref[g_hi, 0] + cs_ref[g_hi, 0]).astype(jnp.int32)
        c_hi = (e_col + BLK - 1) // BLK
        rank = jax.lax.fori_loop(c_lo, c_hi,
                                 col_chunk, jnp.zeros((BLK, 1), jnp.float32))

        mb_r = (bt_r == gi64).astype(jnp.float32)
        k_r = jnp.dot(mb_r, k_s[...], preferred_element_type=jnp.float32)
        keep_r = mk_r * (rank < k_r).astype(jnp.float32)
        st_r = jnp.dot(mb_r, st_ref[0:B, :],
                       preferred_element_type=jnp.float32)
        keep_ref[pl.ds(r0, BLK), :] = keep_r
        pos_ref[pl.ds(r0, BLK), :] = st_r + rank
        cnt_ref[...] += jnp.sum(mb_r * keep_r, axis=0)[:, None]


def _k_topk(sc_col, bt_col, mk_col, pv_col, cs, st):
    full = lambda shape: pl.BlockSpec(shape, lambda i: (0, 0))
    return pl.pallas_call(
        _topk_body,
        grid=(2 * NB,),
        in_specs=[full((NP, 1)), full((NP, 1)), full((NP, 1)), full((NP, 1)),
                  full((1, NP)), full((1, NP)), full((1, NP)), full((1, NP)),
                  full((128, 1)), full((128, 1))],
        out_specs=[full((NP, 1)), full((NP, 1)), full((B, 1))],
        out_shape=[jax.ShapeDtypeStruct((NP, 1), jnp.float32),
                   jax.ShapeDtypeStruct((NP, 1), jnp.float32),
                   jax.ShapeDtypeStruct((B, 1), jnp.float32)],
        scratch_shapes=[pltpu.VMEM((B, 1), jnp.float32),
                        pltpu.VMEM((B, 1), jnp.float32)],
    )(sc_col, bt_col, mk_col, pv_col,
      sc_col.reshape(1, NP), bt_col.reshape(1, NP),
      mk_col.reshape(1, NP), pv_col.reshape(1, NP), cs, st)


def _pool_body(h1_ref, sc_ref, keep_ref, bt_ref, cnt_ref,
               h2_ref, read_ref, sum_s, max_s):
    i = pl.program_id(0)

    @pl.when(i == 0)
    def _():
        sum_s[...] = jnp.zeros_like(sum_s)
        max_s[...] = jnp.full_like(max_s, NEGINF)

    keep = keep_ref[...]
    h2 = h1_ref[...] * (sc_ref[...] * keep)
    h2_ref[:, 0:512] = h2
    li = jax.lax.broadcasted_iota(jnp.int32, (BLK, 128), 1)
    h2_ref[:, 512:640] = jnp.where(li == 0, keep, 0.0)
    bt = bt_ref[...]
    gi64 = jax.lax.broadcasted_iota(jnp.int32, (1, B), 1)
    mk = (bt == gi64).astype(jnp.float32) * keep
    sum_s[...] += jax.lax.dot_general(
        mk, h2, (((0,), (0,)), ((), ())),
        preferred_element_type=jnp.float32)

    g_lo = bt[0, 0]
    g_hi = jnp.minimum(bt[BLK - 1, 0], B - 1)

    def upd(g, _):
        rows = (bt == g) & (keep > 0)
        cand = jnp.max(jnp.where(rows, h2, NEGINF), axis=0, keepdims=True)
        cur = max_s[pl.ds(g, 1), :]
        max_s[pl.ds(g, 1), :] = jnp.maximum(cur, cand)
        return 0

    jax.lax.fori_loop(g_lo, g_hi + 1, upd, 0)

    @pl.when(i == NB - 1)
    def _():
        cnt = cnt_ref[...]
        read_ref[:, 0:512] = jnp.where(cnt > 0, max_s[...], 0.0)
        read_ref[:, 512:1024] = sum_s[...] / jnp.maximum(cnt, 1.0)


def _k_pool(h1, sc, keep, bt, cnt):
    full = lambda shape: pl.BlockSpec(shape, lambda i: (0, 0))
    return pl.pallas_call(
        _pool_body,
        grid=(NB,),
        in_specs=[pl.BlockSpec((BLK, 512), lambda i: (i, 0)),
                  pl.BlockSpec((BLK, 1), lambda i: (i, 0)),
                  pl.BlockSpec((BLK, 1), lambda i: (i, 0)),
                  pl.BlockSpec((BLK, 1), lambda i: (i, 0)),
                  full((B, 1))],
        out_specs=[pl.BlockSpec((BLK, 640), lambda i: (i, 0)),
                   full((B, 1024))],
        out_shape=[jax.ShapeDtypeStruct((NP, 640), jnp.float32),
                   jax.ShapeDtypeStruct((B, 1024), jnp.float32)],
        scratch_shapes=[pltpu.VMEM((B, 512), jnp.float32),
                        pltpu.VMEM((B, 512), jnp.float32)],
    )(h1, sc, keep, bt, cnt)


def _mlp_body(r1_ref, r2_ref, r3_ref, w1, b1, w2, b2, w3, b3, w4, b4, w5, b5,
              o_ref):
    z = r1_ref[...] + r2_ref[...] + r3_ref[...]
    z = jnp.maximum(jnp.dot(z, w1[...], preferred_element_type=jnp.float32)
                    + b1[...], 0.0)
    z = jnp.maximum(jnp.dot(z, w2[...], preferred_element_type=jnp.float32)
                    + b2[...], 0.0)
    z = jnp.maximum(jnp.dot(z, w3[...], preferred_element_type=jnp.float32)
                    + b3[...], 0.0)
    z = jnp.maximum(jnp.dot(z, w4[...], preferred_element_type=jnp.float32)
                    + b4[...], 0.0)
    z = jnp.dot(z, w5[...], preferred_element_type=jnp.float32) + b5[...]
    o_ref[...] = 1.0 / (1.0 + jnp.exp(-z))


def _k_mlp(r1, r2, r3, lw1, lb1, lw2, lb2, lw3, lb3, lw4, lb4, lw5, lb5):
    return pl.pallas_call(
        _mlp_body,
        out_shape=jax.ShapeDtypeStruct((B, 1), jnp.float32),
    )(r1, r2, r3, lw1, lb1[None, :], lw2, lb2[None, :], lw3, lb3[None, :],
      lw4, lb4[None, :], lw5, lb5[None, :])


def kernel(emb, w_l1, b_l1, w_r1, w_l2, b_l2, w_r2, w_l3, b_l3, w_r3,
           p1, p2, p3, lw1, lb1, lw2, lb2, lw3, lb3, lw4, lb4, lw5, lb5,
           x, edge_index, batch):
    ws = {1: (w_l1, b_l1, w_r1, p1), 2: (w_l2, b_l2, w_r2, p2),
          3: (w_l3, b_l3, w_r3, p3)}
    src = edge_index[0].astype(jnp.int32)
    dst = edge_index[1].astype(jnp.int32)
    dst_s, src_s = jax.lax.sort([dst, src], num_keys=1)
    PADE = E + ECH + 8
    src_sp = jnp.pad(src_s, (0, PADE - E))
    dst_sp = jnp.pad(dst_s, (0, PADE - E), constant_values=1 << 20)
    eo1 = jnp.searchsorted(dst_s, jnp.arange(65, dtype=jnp.int32) * 256
                           ).astype(jnp.int32)
    eo1 = jnp.pad(eo1, (0, 128 - 65), constant_values=E)
    eo2 = jnp.searchsorted(dst_s, jnp.arange(97, dtype=jnp.int32) * 128
                           ).astype(jnp.int32)
    eo2 = jnp.pad(eo2, (0, 128 - 97), constant_values=E)

    xi = jnp.pad(x[:, 0].astype(jnp.int32), (0, NP - N))
    h0 = _k_emb(emb, xi)
    keep_col = jnp.pad(jnp.ones((N, 1), jnp.float32), ((0, NP - N), (0, 0)))
    haug = jnp.concatenate([h0, keep_col, jnp.zeros((NP, 127), jnp.float32)],
                           axis=1)
    batchi = jnp.pad(batch.astype(jnp.int32), (0, NP - N),
                     constant_values=B)[:, None]
    cs, st = _k_cnt(batchi)

    maskf = jnp.pad(jnp.ones((N,), jnp.float32), (0, NP - N))[:, None]
    prevf = jnp.arange(NP, dtype=jnp.float32)[:, None]
    keep_full = jnp.ones((NP,), jnp.float32)

    reads = []
    for i in (1, 2, 3):
        w_l, b_l, w_r, p = ws[i]
        din, daug, slc, npass, eo = ((128, 256, 256, 2, eo1) if i == 1
                                     else (512, 640, 128, 3, eo2))
        aggaug = _k_agg(haug, src_sp, dst_sp, keep_full, eo,
                        din, daug, slc, npass)

        h1, sc = _k_dense(haug, aggaug, w_l, b_l, w_r, p, din)
        keepf, posf, cnt = _k_topk(sc, batchi, maskf, prevf, cs, st)
        haug, read = _k_pool(h1, sc, keepf, batchi, cnt)
        reads.append(read)
        maskf, prevf = keepf, posf
        keep_full = keepf[:, 0]

    out = _k_mlp(reads[0], reads[1], reads[2],
                 lw1, lb1, lw2, lb2, lw3, lb3, lw4, lb4, lw5, lb5)
    return out[:, 0]


# speedup vs baseline: 7.1635x; 1.5599x over previous
import functools
import jax
import jax.numpy as jnp
from jax import lax
from jax.experimental import pallas as pl
from jax.experimental.pallas import tpu as pltpu
from jax.experimental.pallas import tpu_sc as plsc

N = 10000
NP = 10240
BLK = 1024
NB = NP // BLK
B = 64
RATIO = 0.8
NEGINF = -jnp.inf

NC, NS, L = 2, 16, 16
E = 320000
EPT = E // NS
ECH = 2000
NCHUNK = EPT // ECH
GCAP = EPT + 160


def _sc_mesh():
    return plsc.VectorSubcoreMesh(core_axis_name="c", subcore_axis_name="s",
                                  num_cores=NC, num_subcores=NS)


def _emb_body(emb_hbm, xi_hbm, out_hbm, idx_v, rows_v, sem):
    c = lax.axis_index("c")
    s = lax.axis_index("s")
    wid = s * NC + c
    bpw = NP // (NC * NS)
    base = wid * bpw
    pltpu.sync_copy(xi_hbm.at[pl.ds(base, bpw)], idx_v)
    for k in range(bpw // 64):
        pltpu.async_copy(emb_hbm.at[idx_v.at[pl.ds(k * 64, 64)]],
                         rows_v.at[pl.ds(k * 64, 64)], sem).wait()
    pltpu.sync_copy(rows_v, out_hbm.at[pl.ds(base, bpw)])


def _k_emb(emb, xi):
    bpw = NP // (NC * NS)
    return pl.kernel(
        _emb_body,
        out_type=jax.ShapeDtypeStruct((NP, 128), jnp.float32),
        mesh=_sc_mesh(),
        scratch_types=[pltpu.VMEM((bpw,), jnp.int32),
                       pltpu.VMEM((bpw, 128), jnp.float32),
                       pltpu.SemaphoreType.DMA],
    )(emb, xi)


def _make_agg_body(din, daug, slice_rows, npass):
    def body(h_hbm, src_hbm, dst_hbm, keep_hbm, eo_hbm, agg_hbm,
             keep_v, eo_v, src_v, dst_v, gidx, sidx, rows_v, acc_v, sem):
        c = lax.axis_index("c")
        s = lax.axis_index("s")
        wid = s * NC + c
        pltpu.sync_copy(keep_hbm, keep_v)
        pltpu.sync_copy(eo_hbm, eo_v)

        for p in range(npass):
            k = p * (NC * NS) + wid
            base = k * slice_rows

            def zr(r, _):
                for cc in range(daug // L):
                    acc_v[r, pl.ds(cc * L, L)] = jnp.zeros((L,), jnp.float32)
                return 0
            lax.fori_loop(0, slice_rows, zr, 0)

            eo16 = eo_v[pl.ds(k, L)]
            e0 = eo16[0]
            e1 = eo16[1]
            a0 = (e0 // 8) * 8
            nch = (e1 - a0 + ECH - 1) // ECH

            def chunk(ch, carry):
                cb = a0 + ch * ECH
                pltpu.sync_copy(src_hbm.at[pl.ds(cb, ECH)], src_v)
                pltpu.sync_copy(dst_hbm.at[pl.ds(cb, ECH)], dst_v)

                def scan16(j, cnt):
                    s16 = src_v[pl.ds(j * L, L)]
                    d16 = dst_v[pl.ds(j * L, L)]
                    ks = plsc.load_gather(keep_v, [s16])
                    kd = plsc.load_gather(keep_v, [d16])
                    m = ((d16 >= base) & (d16 < base + slice_rows)
                         & (ks > 0.0) & (kd > 0.0))
                    plsc.store_compressed(gidx.at[pl.ds(cnt, L)], s16, mask=m)
                    plsc.store_compressed(sidx.at[pl.ds(cnt, L)], d16 - base,
                                          mask=m)
                    return cnt + jnp.max(
                        plsc.all_reduce_population_count(m))
                cnt = lax.fori_loop(0, ECH // L, scan16, jnp.int32(0))

                for t in range(2):
                    gidx[pl.ds(cnt + t * L, L)] = jnp.zeros((L,), jnp.int32)
                nsg = (cnt + 31) // 32

                def fl(sg, carry):
                    cps = []
                    for j in range(2):
                        i16 = gidx[pl.ds((sg * 2 + j) * L, L)]
                        cps.append(pltpu.async_copy(
                            h_hbm.at[i16], rows_v.at[pl.ds(j * L, L)], sem))
                    for cp in cps:
                        cp.wait()

                    def acc_row(r, carry):
                        prev, accs = carry
                        slot = r - sg * 32
                        dl = sidx[pl.ds(r, L)][0]
                        same = dl == prev
                        row = [rows_v[slot, pl.ds(cc * L, L)]
                               for cc in range(daug // L)]
                        accs = [jnp.where(same, a + x, x)
                                for a, x in zip(accs, row)]
                        for cc in range(daug // L):
                            acc_v[dl, pl.ds(cc * L, L)] = accs[cc]
                        return (dl, accs)
                    return lax.fori_loop(sg * 32,
                                         jnp.minimum((sg + 1) * 32, cnt),
                                         acc_row, carry)
                return lax.fori_loop(0, nsg, fl, carry)
            carry0 = (jnp.int32(-1),
                      [jnp.zeros((L,), jnp.float32)] * (daug // L))
            lax.fori_loop(0, nch, chunk, carry0)

            @pl.when(base < NP)
            def _():
                pltpu.sync_copy(acc_v,
                                agg_hbm.at[pl.ds(base, slice_rows)])
    return body


def _k_agg(haug, src_s, dst_s, keepf, eo, din, daug, slice_rows, npass):
    body = _make_agg_body(din, daug, slice_rows, npass)
    return pl.kernel(
        body,
        out_type=jax.ShapeDtypeStruct((NP, daug), jnp.float32),
        mesh=_sc_mesh(),
        compiler_params=pltpu.CompilerParams(needs_layout_passes=False),
        scratch_types=[pltpu.VMEM((NP,), jnp.float32),
                       pltpu.VMEM((128,), jnp.int32),
                       pltpu.VMEM((ECH,), jnp.int32),
                       pltpu.VMEM((ECH,), jnp.int32),
                       pltpu.VMEM((ECH + 64,), jnp.int32),
                       pltpu.VMEM((ECH + 64,), jnp.int32),
                       pltpu.VMEM((32, daug), jnp.float32),
                       pltpu.VMEM((slice_rows, daug), jnp.float32),
                       pltpu.SemaphoreType.DMA],
    )(haug, src_s, dst_s, keepf, eo)


def _cnt_body(bt_ref, cs_ref, st_ref):
    i = pl.program_id(0)

    @pl.when(i == 0)
    def _():
        cs_ref[...] = jnp.zeros_like(cs_ref)

    bt = bt_ref[pl.ds(i * BLK, BLK), :]
    gi = jax.lax.broadcasted_iota(jnp.int32, (1, 128), 1)
    mb = (bt == gi).astype(jnp.float32)
    cs_ref[...] += jnp.sum(mb, axis=0)[:, None]

    @pl.when(i == NB - 1)
    def _():
        r = jax.lax.broadcasted_iota(jnp.int32, (128, 128), 0)
        c = jax.lax.broadcasted_iota(jnp.int32, (128, 128), 1)
        ltri = (c < r).astype(jnp.float32)
        st_ref[...] = jnp.dot(ltri, cs_ref[...],
                              preferred_element_type=jnp.float32)


def _k_cnt(batchi):
    return pl.pallas_call(
        _cnt_body,
        grid=(NB,),
        in_specs=[pl.BlockSpec((NP, 1), lambda i: (0, 0))],
        out_specs=[pl.BlockSpec((128, 1), lambda i: (0, 0)),
                   pl.BlockSpec((128, 1), lambda i: (0, 0))],
        out_shape=[jax.ShapeDtypeStruct((128, 1), jnp.float32),
                   jax.ShapeDtypeStruct((128, 1), jnp.float32)],
    )(batchi)


def _dense_body(din, h_ref, agg_ref, wl_ref, bl_ref, wr_ref, p_ref,
                h1_ref, sc_ref):
    agg = agg_ref[:, 0:din]
    deg = agg_ref[:, din:din + 1]
    a = agg / jnp.maximum(deg, 1.0)
    h1 = jnp.dot(a, wl_ref[...], preferred_element_type=jnp.float32)
    h1 += jnp.dot(h_ref[:, 0:din], wr_ref[...],
                  preferred_element_type=jnp.float32)
    h1 = jnp.maximum(h1 + bl_ref[...], 0.0)
    h1_ref[...] = h1
    pv = p_ref[...]
    nrm = jnp.sqrt(jnp.sum(pv * pv))
    sc_ref[...] = jnp.tanh(
        jnp.dot(h1, pv, preferred_element_type=jnp.float32) / nrm)


def _k_dense(haug, aggaug, w_l, b_l, w_r, p, din):
    daug = din + 128
    return pl.pallas_call(
        functools.partial(_dense_body, din),
        grid=(NB,),
        in_specs=[
            pl.BlockSpec((BLK, daug), lambda i: (i, 0)),
            pl.BlockSpec((BLK, daug), lambda i: (i, 0)),
            pl.BlockSpec((din, 512), lambda i: (0, 0)),
            pl.BlockSpec((1, 512), lambda i: (0, 0)),
            pl.BlockSpec((din, 512), lambda i: (0, 0)),
            pl.BlockSpec((512, 1), lambda i: (0, 0)),
        ],
        out_specs=[pl.BlockSpec((BLK, 512), lambda i: (i, 0)),
                   pl.BlockSpec((BLK, 1), lambda i: (i, 0))],
        out_shape=[jax.ShapeDtypeStruct((NP, 512), jnp.float32),
                   jax.ShapeDtypeStruct((NP, 1), jnp.float32)],
    )(haug, aggaug, w_l, b_l[None, :], w_r, p[:, None])


def _topk_body(sc_col_ref, bt_col_ref, mk_col_ref, pv_col_ref,
               sc_row_ref, bt_row_ref, mk_row_ref, pv_row_ref,
               cs_ref, st_ref,
               keep_ref, pos_ref, cnt_ref,
               kept_s, k_s):
    pid = pl.program_id(0)
    gi64 = jax.lax.broadcasted_iota(jnp.int32, (1, B), 1)

    @pl.when(pid == 0)
    def _():
        kept_s[...] = jnp.zeros_like(kept_s)

    @pl.when(pid < NB)
    def _():
        bt = bt_col_ref[pl.ds(pid * BLK, BLK), :]
        m = mk_col_ref[pl.ds(pid * BLK, BLK), :]
        mb = (bt == gi64).astype(jnp.float32)
        kept_s[...] += jnp.sum(mb * m, axis=0)[:, None]

    @pl.when(pid == NB - 1)
    def _():
        k_s[...] = jnp.ceil(jnp.float32(RATIO) * kept_s[...])

    @pl.when(pid == NB)
    def _():
        cnt_ref[...] = jnp.zeros_like(cnt_ref)

    @pl.when(pid >= NB)
    def _():
        i2 = pid - NB
        r0 = i2 * BLK
        bt_r = bt_col_ref[pl.ds(r0, BLK), :]
        mk_r = mk_col_ref[pl.ds(r0, BLK), :]
        sc_r = sc_col_ref[pl.ds(r0, BLK), :]
        pv_r = pv_col_ref[pl.ds(r0, BLK), :]
        neg_r = jnp.where(mk_r > 0, -sc_r, jnp.inf)

        def col_chunk(c, rank):
            bt_c = bt_row_ref[:, pl.ds(c * BLK, BLK)]
            mk_c = mk_row_ref[:, pl.ds(c * BLK, BLK)]
            sc_c = sc_row_ref[:, pl.ds(c * BLK, BLK)]
            pv_c = pv_row_ref[:, pl.ds(c * BLK, BLK)]
            neg_c = jnp.where(mk_c > 0, -sc_c, jnp.inf)
            same = bt_c == bt_r
            lt = (neg_c < neg_r) | ((neg_c == neg_r) & (pv_c < pv_r))
            return rank + jnp.sum((same & lt).astype(jnp.float32),
                                  axis=1, keepdims=True)

        g_lo = bt_col_ref[r0, 0]
        g_hi = bt_col_ref[r0 + BLK - 1, 0]
        c_lo = st_ref[g_lo, 0].astype(jnp.int32) // BLK
        e_col = (st_ref[g_hi, 0] + cs_ref[g_hi, 0]).astype(jnp.int32)
        c_hi = (e_col + BLK - 1) // BLK
        rank = jax.lax.fori_loop(c_lo, c_hi,
                                 col_chunk, jnp.zeros((BLK, 1), jnp.float32))

        mb_r = (bt_r == gi64).astype(jnp.float32)
        k_r = jnp.dot(mb_r, k_s[...], preferred_element_type=jnp.float32)
        keep_r = mk_r * (rank < k_r).astype(jnp.float32)
        st_r = jnp.dot(mb_r, st_ref[0:B, :],
                       preferred_element_type=jnp.float32)
        keep_ref[pl.ds(r0, BLK), :] = keep_r
        pos_ref[pl.ds(r0, BLK), :] = st_r + rank
        cnt_ref[...] += jnp.sum(mb_r * keep_r, axis=0)[:, None]


def _k_topk(sc_col, bt_col, mk_col, pv_col, cs, st):
    full = lambda shape: pl.BlockSpec(shape, lambda i: (0, 0))
    return pl.pallas_call(
        _topk_body,
        grid=(2 * NB,),
        in_specs=[full((NP, 1)), full((NP, 1)), full((NP, 1)), full((NP, 1)),
                  full((1, NP)), full((1, NP)), full((1, NP)), full((1, NP)),
                  full((128, 1)), full((128, 1))],
        out_specs=[full((NP, 1)), full((NP, 1)), full((B, 1))],
        out_shape=[jax.ShapeDtypeStruct((NP, 1), jnp.float32),
                   jax.ShapeDtypeStruct((NP, 1), jnp.float32),
                   jax.ShapeDtypeStruct((B, 1), jnp.float32)],
        scratch_shapes=[pltpu.VMEM((B, 1), jnp.float32),
                        pltpu.VMEM((B, 1), jnp.float32)],
    )(sc_col, bt_col, mk_col, pv_col,
      sc_col.reshape(1, NP), bt_col.reshape(1, NP),
      mk_col.reshape(1, NP), pv_col.reshape(1, NP), cs, st)


def _pool_body(h1_ref, sc_ref, keep_ref, bt_ref, cnt_ref,
               h2_ref, read_ref, sum_s, max_s):
    i = pl.program_id(0)

    @pl.when(i == 0)
    def _():
        sum_s[...] = jnp.zeros_like(sum_s)
        max_s[...] = jnp.full_like(max_s, NEGINF)

    keep = keep_ref[...]
    h2 = h1_ref[...] * (sc_ref[...] * keep)
    h2_ref[:, 0:512] = h2
    li = jax.lax.broadcasted_iota(jnp.int32, (BLK, 128), 1)
    h2_ref[:, 512:640] = jnp.where(li == 0, keep, 0.0)
    bt = bt_ref[...]
    gi64 = jax.lax.broadcasted_iota(jnp.int32, (1, B), 1)
    mk = (bt == gi64).astype(jnp.float32) * keep
    sum_s[...] += jax.lax.dot_general(
        mk, h2, (((0,), (0,)), ((), ())),
        preferred_element_type=jnp.float32)

    g_lo = bt[0, 0]
    g_hi = jnp.minimum(bt[BLK - 1, 0], B - 1)

    def upd(g, _):
        rows = (bt == g) & (keep > 0)
        cand = jnp.max(jnp.where(rows, h2, NEGINF), axis=0, keepdims=True)
        cur = max_s[pl.ds(g, 1), :]
        max_s[pl.ds(g, 1), :] = jnp.maximum(cur, cand)
        return 0

    jax.lax.fori_loop(g_lo, g_hi + 1, upd, 0)

    @pl.when(i == NB - 1)
    def _():
        cnt = cnt_ref[...]
        read_ref[:, 0:512] = jnp.where(cnt > 0, max_s[...], 0.0)
        read_ref[:, 512:1024] = sum_s[...] / jnp.maximum(cnt, 1.0)


def _k_pool(h1, sc, keep, bt, cnt):
    full = lambda shape: pl.BlockSpec(shape, lambda i: (0, 0))
    return pl.pallas_call(
        _pool_body,
        grid=(NB,),
        in_specs=[pl.BlockSpec((BLK, 512), lambda i: (i, 0)),
                  pl.BlockSpec((BLK, 1), lambda i: (i, 0)),
                  pl.BlockSpec((BLK, 1), lambda i: (i, 0)),
                  pl.BlockSpec((BLK, 1), lambda i: (i, 0)),
                  full((B, 1))],
        out_specs=[pl.BlockSpec((BLK, 640), lambda i: (i, 0)),
                   full((B, 1024))],
        out_shape=[jax.ShapeDtypeStruct((NP, 640), jnp.float32),
                   jax.ShapeDtypeStruct((B, 1024), jnp.float32)],
        scratch_shapes=[pltpu.VMEM((B, 512), jnp.float32),
                        pltpu.VMEM((B, 512), jnp.float32)],
    )(h1, sc, keep, bt, cnt)


def _mlp_body(r1_ref, r2_ref, r3_ref, w1, b1, w2, b2, w3, b3, w4, b4, w5, b5,
              o_ref):
    z = r1_ref[...] + r2_ref[...] + r3_ref[...]
    z = jnp.maximum(jnp.dot(z, w1[...], preferred_element_type=jnp.float32)
                    + b1[...], 0.0)
    z = jnp.maximum(jnp.dot(z, w2[...], preferred_element_type=jnp.float32)
                    + b2[...], 0.0)
    z = jnp.maximum(jnp.dot(z, w3[...], preferred_element_type=jnp.float32)
                    + b3[...], 0.0)
    z = jnp.maximum(jnp.dot(z, w4[...], preferred_element_type=jnp.float32)
                    + b4[...], 0.0)
    z = jnp.dot(z, w5[...], preferred_element_type=jnp.float32) + b5[...]
    o_ref[...] = 1.0 / (1.0 + jnp.exp(-z))


def _k_mlp(r1, r2, r3, lw1, lb1, lw2, lb2, lw3, lb3, lw4, lb4, lw5, lb5):
    return pl.pallas_call(
        _mlp_body,
        out_shape=jax.ShapeDtypeStruct((B, 1), jnp.float32),
    )(r1, r2, r3, lw1, lb1[None, :], lw2, lb2[None, :], lw3, lb3[None, :],
      lw4, lb4[None, :], lw5, lb5[None, :])


def kernel(emb, w_l1, b_l1, w_r1, w_l2, b_l2, w_r2, w_l3, b_l3, w_r3,
           p1, p2, p3, lw1, lb1, lw2, lb2, lw3, lb3, lw4, lb4, lw5, lb5,
           x, edge_index, batch):
    ws = {1: (w_l1, b_l1, w_r1, p1), 2: (w_l2, b_l2, w_r2, p2),
          3: (w_l3, b_l3, w_r3, p3)}
    src = edge_index[0].astype(jnp.int32)
    dst = edge_index[1].astype(jnp.int32)
    dst_s, src_s = jax.lax.sort([dst, src], num_keys=1)
    PADE = E + ECH + 8
    src_sp = jnp.pad(src_s, (0, PADE - E))
    dst_sp = jnp.pad(dst_s, (0, PADE - E), constant_values=1 << 20)
    eo1 = jnp.searchsorted(dst_s, jnp.arange(65, dtype=jnp.int32) * 256
                           ).astype(jnp.int32)
    eo1 = jnp.pad(eo1, (0, 128 - 65), constant_values=E)
    eo2 = jnp.searchsorted(dst_s, jnp.arange(97, dtype=jnp.int32) * 128
                           ).astype(jnp.int32)
    eo2 = jnp.pad(eo2, (0, 128 - 97), constant_values=E)

    xi = jnp.pad(x[:, 0].astype(jnp.int32), (0, NP - N))
    h0 = _k_emb(emb, xi)
    keep_col = jnp.pad(jnp.ones((N, 1), jnp.float32), ((0, NP - N), (0, 0)))
    haug = jnp.concatenate([h0, keep_col, jnp.zeros((NP, 127), jnp.float32)],
                           axis=1)
    batchi = jnp.pad(batch.astype(jnp.int32), (0, NP - N),
                     constant_values=B)[:, None]
    cs, st = _k_cnt(batchi)

    maskf = jnp.pad(jnp.ones((N,), jnp.float32), (0, NP - N))[:, None]
    prevf = jnp.arange(NP, dtype=jnp.float32)[:, None]
    keep_full = jnp.ones((NP,), jnp.float32)

    reads = []
    for i in (1, 2, 3):
        w_l, b_l, w_r, p = ws[i]
        din, daug, slc, npass, eo = ((128, 256, 256, 2, eo1) if i == 1
                                     else (512, 640, 128, 3, eo2))
        aggaug = _k_agg(haug, src_sp, dst_sp, keep_full, eo,
                        din, daug, slc, npass)

        h1, sc = _k_dense(haug, aggaug, w_l, b_l, w_r, p, din)
        keepf, posf, cnt = _k_topk(sc, batchi, maskf, prevf, cs, st)
        haug, read = _k_pool(h1, sc, keepf, batchi, cnt)
        reads.append(read)
        maskf, prevf = keepf, posf
        keep_full = keepf[:, 0]

    out = _k_mlp(reads[0], reads[1], reads[2],
                 lw1, lb1, lw2, lb2, lw3, lb3, lw4, lb4, lw5, lb5)
    return out[:, 0]
